# baseline (speedup 1.0000x reference)
import functools

import jax
import jax.numpy as jnp
from jax import lax
from jax.experimental import pallas as pl
from jax.experimental.pallas import tpu as pltpu
from jax.experimental.pallas import tpu_sc as plsc

VOCAB = 1000000
STYLE = 18
B = 16384
L = 200
EPS = 1e-12

N = B * L
NW = 32
TOK_PER_W = N // NW
CHUNK = 1024
GATHER = 128
N_GATHER = 2 * CHUNK // GATHER
GROUPS = CHUNK // 16
N_CHUNK = TOK_PER_W // CHUNK
N_PAIR = N_CHUNK // 2
WROWS = VOCAB * STYLE // 16


def _rsqrt(v):
    i = plsc.bitcast(v, jnp.int32)
    i = jnp.int32(0x5F3759DF) - (i >> 1)
    y = plsc.bitcast(i, jnp.float32)
    for _ in range(3):
        y = y * (1.5 - 0.5 * v * y * y)
    return y


def _body(sidx_hbm, wids_hbm, bids_hbm, bot_hbm, word_hbm, gam_hbm, bet_hbm,
          out_hbm,
          sidx_v, wid_v, bid_v, rows_v, out_v, bot_v, gam_v, bet_v,
          sem_a, sem_b):
    nc = 2
    w = lax.axis_index("s") * nc + lax.axis_index("c")
    base_w = w * TOK_PER_W

    pltpu.sync_copy(bot_hbm, bot_v)
    pltpu.sync_copy(gam_hbm, gam_v)
    pltpu.sync_copy(bet_hbm, bet_v)
    g0, g1 = gam_v[pl.ds(0, 16)], gam_v[pl.ds(16, 16)]
    b0, b1 = bet_v[pl.ds(0, 16)], bet_v[pl.ds(16, 16)]
    gam = [g0[d] for d in range(16)] + [g1[0], g1[1]]
    bet = [b0[d] for d in range(16)] + [b1[0], b1[1]]

    lanes = lax.broadcasted_iota(jnp.int32, (16,), 0)
    sem = [sem_a, sem_b]

    def gather_copy(ci, p, j):
        return pltpu.make_async_copy(
            word_hbm.at[sidx_v.at[p, j]],
            rows_v.at[p, pl.ds(j * GATHER, GATHER), :],
            sem[p])

    def stage(ci, p):
        tok = base_w + ci * CHUNK
        row0 = pl.multiple_of(tok // 64, 16)
        pltpu.sync_copy(sidx_hbm.at[pl.ds(row0, N_GATHER), :],
                        sidx_v.at[p])
        pltpu.sync_copy(wids_hbm.at[pl.ds(tok, CHUNK)], wid_v.at[p])
        pltpu.sync_copy(bids_hbm.at[pl.ds(tok, CHUNK)], bid_v.at[p])
        for j in range(N_GATHER):
            gather_copy(ci, p, j).start()

    def process(ci, p):
        for j in range(N_GATHER):
            gather_copy(ci, p, j).wait()

        def group_body(g, _):
            tvec = lanes + g * 16
            widv = wid_v[p, pl.ds(g * 16, 16)]
            bidv = bid_v[p, pl.ds(g * 16, 16)] * STYLE
            base = tvec * 32 + ((widv * STYLE) & 15)
            x = []
            for d in range(STYLE):
                fi = base + d
                wv = plsc.load_gather(rows_v.at[p], [fi >> 4, fi & 15])
                bv = plsc.load_gather(bot_v, [bidv + d])
                x.append(wv * bv)
            s = x[0]
            for d in range(1, STYLE):
                s = s + x[d]
            m = s * (1.0 / STYLE)
            t = [xd - m for xd in x]
            q = t[0] * t[0]
            for d in range(1, STYLE):
                q = q + t[d] * t[d]
            r = _rsqrt(q * (1.0 / STYLE) + EPS)
            oidx = tvec * STYLE
            for d in range(STYLE):
                yd = t[d] * (r * gam[d]) + bet[d]
                plsc.store_scatter(out_v.at[p], [oidx + d], yd)
            return None

        lax.fori_loop(0, GROUPS, group_body, None)
        tok = base_w + ci * CHUNK
        pltpu.sync_copy(out_v.at[p],
                        out_hbm.at[pl.ds(tok * STYLE, CHUNK * STYLE)])

    stage(0, 0)

    def pair_body(k, _):
        stage(2 * k + 1, 1)
        process(2 * k, 0)

        @pl.when(k < N_PAIR - 1)
        def _():
            stage(2 * k + 2, 0)

        process(2 * k + 1, 1)
        return None

    lax.fori_loop(0, N_PAIR, pair_body, None)


@jax.jit
def _run(sidx, wids, bids, bottom_flat, word16, gamma32, beta32):
    mesh = plsc.VectorSubcoreMesh(core_axis_name="c", subcore_axis_name="s")
    f = functools.partial(
        pl.kernel,
        mesh=mesh,
        out_type=jax.ShapeDtypeStruct((N * STYLE,), jnp.float32),
        scratch_types=[
            pltpu.VMEM((2, N_GATHER, GATHER), jnp.int32),
            pltpu.VMEM((2, CHUNK), jnp.int32),
            pltpu.VMEM((2, CHUNK), jnp.int32),
            pltpu.VMEM((2, 2 * CHUNK, 16), jnp.float32),
            pltpu.VMEM((2, CHUNK * STYLE), jnp.float32),
            pltpu.VMEM((STYLE * STYLE,), jnp.float32),
            pltpu.VMEM((32,), jnp.float32),
            pltpu.VMEM((32,), jnp.float32),
            pltpu.SemaphoreType.DMA,
            pltpu.SemaphoreType.DMA,
        ],
        compiler_params=pltpu.CompilerParams(
            needs_layout_passes=False, use_tc_tiling_on_sc=False),
    )(_body)
    return f(sidx, wids, bids, bottom_flat, word16, gamma32, beta32)


def kernel(input_bids, input_wids, bottom_emb, word_emb, gamma, beta):
    wids = input_wids.reshape(-1).astype(jnp.int32)
    bids = input_bids.reshape(-1).astype(jnp.int32)
    blk0 = (wids * STYLE) >> 4
    sidx = jnp.stack([blk0, blk0 + 1], axis=-1).reshape(-1, GATHER)
    word16 = word_emb.reshape(WROWS, 16)
    bottom_flat = bottom_emb.reshape(-1)
    gam32 = jnp.zeros((32,), jnp.float32).at[:STYLE].set(gamma)
    bet32 = jnp.zeros((32,), jnp.float32).at[:STYLE].set(beta)
    out = _run(sidx, wids, bids, bottom_flat, word16, gam32, bet32)
    return out.reshape(B, L, STYLE)


# speedup vs baseline: 1.3914x; 1.3914x over previous
import functools

import jax
import jax.numpy as jnp
from jax import lax
from jax.experimental import pallas as pl
from jax.experimental.pallas import tpu as pltpu
from jax.experimental.pallas import tpu_sc as plsc

VOCAB = 1000000
STYLE = 18
WPAD = 128
B = 16384
L = 200
EPS = 1e-12

N = B * L
NW = 32
TOK_PER_W = N // NW
CHUNK = 256
GROUPS = CHUNK // 16
N_CHUNK = TOK_PER_W // CHUNK
N_PAIR = N_CHUNK // 2
PAD_BLK = 1000


def _rsqrt(v):
    i = plsc.bitcast(v, jnp.int32)
    i = jnp.int32(0x5F3759DF) - (i >> 1)
    y = plsc.bitcast(i, jnp.float32)
    for _ in range(3):
        y = y * (1.5 - 0.5 * v * y * y)
    return y


def _pad_body(x_ref, o_ref):
    o_ref[...] = jnp.pad(x_ref[...], ((0, 0), (0, WPAD - STYLE)))


def _pad_table(word_emb):
    return pl.pallas_call(
        _pad_body,
        grid=(VOCAB // PAD_BLK,),
        in_specs=[pl.BlockSpec((PAD_BLK, STYLE), lambda i: (i, 0))],
        out_specs=pl.BlockSpec((PAD_BLK, WPAD), lambda i: (i, 0)),
        out_shape=jax.ShapeDtypeStruct((VOCAB, WPAD), jnp.float32),
    )(word_emb)


def _body(wids_hbm, bids_hbm, bot_hbm, word_hbm, gam_hbm, bet_hbm, out_hbm,
          wid_v, bid_v, rows_v, out_v, bot_v, gam_v, bet_v, sem_a, sem_b):
    nc = 2
    w = lax.axis_index("s") * nc + lax.axis_index("c")
    base_w = w * TOK_PER_W

    pltpu.sync_copy(bot_hbm, bot_v)
    pltpu.sync_copy(gam_hbm, gam_v)
    pltpu.sync_copy(bet_hbm, bet_v)
    g0, g1 = gam_v[pl.ds(0, 16)], gam_v[pl.ds(16, 16)]
    b0, b1 = bet_v[pl.ds(0, 16)], bet_v[pl.ds(16, 16)]
    gam = [g0[d] for d in range(16)] + [g1[0], g1[1]]
    bet = [b0[d] for d in range(16)] + [b1[0], b1[1]]

    lanes = lax.broadcasted_iota(jnp.int32, (16,), 0)
    sem = [sem_a, sem_b]

    def gather_copy(p, i):
        widv = wid_v[p, pl.ds(i * 16, 16)]
        return pltpu.make_async_copy(
            word_hbm.at[widv],
            rows_v.at[p, pl.ds(i * 16, 16), :],
            sem[p])

    def stage(ci, p):
        tok = base_w + ci * CHUNK
        pltpu.sync_copy(wids_hbm.at[pl.ds(tok, CHUNK)], wid_v.at[p])
        pltpu.sync_copy(bids_hbm.at[pl.ds(tok, CHUNK)], bid_v.at[p])
        for i in range(CHUNK // 16):
            gather_copy(p, i).start()

    def process(ci, p):
        for i in range(CHUNK // 16):
            gather_copy(p, i).wait()

        def group_body(g, _):
            tvec = lanes + g * 16
            bidv = bid_v[p, pl.ds(g * 16, 16)] * STYLE
            x = []
            for d in range(STYLE):
                cd = jnp.full((16,), d, jnp.int32)
                wv = plsc.load_gather(rows_v.at[p], [tvec, cd])
                bv = plsc.load_gather(bot_v, [bidv + d])
                x.append(wv * bv)
            s = x[0]
            for d in range(1, STYLE):
                s = s + x[d]
            m = s * (1.0 / STYLE)
            t = [xd - m for xd in x]
            q = t[0] * t[0]
            for d in range(1, STYLE):
                q = q + t[d] * t[d]
            r = _rsqrt(q * (1.0 / STYLE) + EPS)
            oidx = tvec * STYLE
            for d in range(STYLE):
                yd = t[d] * (r * gam[d]) + bet[d]
                plsc.store_scatter(out_v.at[p], [oidx + d], yd)
            return None

        lax.fori_loop(0, GROUPS, group_body, None)
        tok = base_w + ci * CHUNK
        pltpu.sync_copy(out_v.at[p],
                        out_hbm.at[pl.ds(tok * STYLE, CHUNK * STYLE)])

    stage(0, 0)

    def pair_body(k, _):
        stage(2 * k + 1, 1)
        process(2 * k, 0)

        @pl.when(k < N_PAIR - 1)
        def _():
            stage(2 * k + 2, 0)

        process(2 * k + 1, 1)
        return None

    lax.fori_loop(0, N_PAIR, pair_body, None)


@jax.jit
def _run(wids, bids, bottom_flat, word_pad, gamma32, beta32):
    mesh = plsc.VectorSubcoreMesh(core_axis_name="c", subcore_axis_name="s")
    f = functools.partial(
        pl.kernel,
        mesh=mesh,
        out_type=jax.ShapeDtypeStruct((N * STYLE,), jnp.float32),
        scratch_types=[
            pltpu.VMEM((2, CHUNK), jnp.int32),
            pltpu.VMEM((2, CHUNK), jnp.int32),
            pltpu.VMEM((2, CHUNK, WPAD), jnp.float32),
            pltpu.VMEM((2, CHUNK * STYLE), jnp.float32),
            pltpu.VMEM((STYLE * STYLE,), jnp.float32),
            pltpu.VMEM((32,), jnp.float32),
            pltpu.VMEM((32,), jnp.float32),
            pltpu.SemaphoreType.DMA,
            pltpu.SemaphoreType.DMA,
        ],
        compiler_params=pltpu.CompilerParams(
            needs_layout_passes=False, use_tc_tiling_on_sc=False),
    )(_body)
    return f(wids, bids, bottom_flat, word_pad, gamma32, beta32)


def kernel(input_bids, input_wids, bottom_emb, word_emb, gamma, beta):
    wids = input_wids.reshape(-1).astype(jnp.int32)
    bids = input_bids.reshape(-1).astype(jnp.int32)
    word_pad = _pad_table(word_emb)
    bottom_flat = bottom_emb.reshape(-1)
    gam32 = jnp.zeros((32,), jnp.float32).at[:STYLE].set(gamma)
    bet32 = jnp.zeros((32,), jnp.float32).at[:STYLE].set(beta)
    out = _run(wids, bids, bottom_flat, word_pad, gam32, bet32)
    return out.reshape(B, L, STYLE)


# speedup vs baseline: 1.4848x; 1.0672x over previous
import functools

import jax
import jax.numpy as jnp
from jax import lax
from jax.experimental import pallas as pl
from jax.experimental.pallas import tpu as pltpu
from jax.experimental.pallas import tpu_sc as plsc

VOCAB = 1000000
STYLE = 18
WPAD = 24
B = 16384
L = 200
EPS = 1e-12

N = B * L
NW = 32
TOK_PER_W = N // NW
CHUNK = 1024
GROUPS = CHUNK // 16
N_CHUNK = TOK_PER_W // CHUNK
N_PAIR = N_CHUNK // 2
PAD_BLK = 1000


def _rsqrt(v):
    i = plsc.bitcast(v, jnp.int32)
    i = jnp.int32(0x5F3759DF) - (i >> 1)
    y = plsc.bitcast(i, jnp.float32)
    for _ in range(3):
        y = y * (1.5 - 0.5 * v * y * y)
    return y


def _pad_body(x_ref, o_ref):
    o_ref[...] = jnp.pad(x_ref[...], ((0, 0), (0, WPAD - STYLE)))


def _pad_table(word_emb):
    return pl.pallas_call(
        _pad_body,
        grid=(VOCAB // PAD_BLK,),
        in_specs=[pl.BlockSpec((PAD_BLK, STYLE), lambda i: (i, 0))],
        out_specs=pl.BlockSpec((PAD_BLK, WPAD), lambda i: (i, 0)),
        out_shape=jax.ShapeDtypeStruct((VOCAB, WPAD), jnp.float32),
    )(word_emb)


def _body(wids_hbm, bids_hbm, bot_hbm, word_hbm, gam_hbm, bet_hbm, out_hbm,
          wid_v, bid_v, rows_v, out_v, bot_v, gam_v, bet_v, sem_a, sem_b):
    nc = 2
    w = lax.axis_index("s") * nc + lax.axis_index("c")
    base_w = w * TOK_PER_W

    pltpu.sync_copy(bot_hbm, bot_v)
    pltpu.sync_copy(gam_hbm, gam_v)
    pltpu.sync_copy(bet_hbm, bet_v)
    g0, g1 = gam_v[pl.ds(0, 16)], gam_v[pl.ds(16, 16)]
    b0, b1 = bet_v[pl.ds(0, 16)], bet_v[pl.ds(16, 16)]
    gam = [g0[d] for d in range(16)] + [g1[0], g1[1]]
    bet = [b0[d] for d in range(16)] + [b1[0], b1[1]]

    lanes = lax.broadcasted_iota(jnp.int32, (16,), 0)
    sem = [sem_a, sem_b]

    def gather_copy(p, i):
        widv = wid_v[p, pl.ds(i * 16, 16)]
        return pltpu.make_async_copy(
            word_hbm.at[widv],
            rows_v.at[p, pl.ds(i * 16, 16), :],
            sem[p])

    def stage(ci, p):
        tok = base_w + ci * CHUNK
        pltpu.sync_copy(wids_hbm.at[pl.ds(tok, CHUNK)], wid_v.at[p])
        pltpu.sync_copy(bids_hbm.at[pl.ds(tok, CHUNK)], bid_v.at[p])
        for i in range(CHUNK // 16):
            gather_copy(p, i).start()

    def process(ci, p):
        for i in range(CHUNK // 16):
            gather_copy(p, i).wait()

        def group_body(g, _):
            tvec = lanes + g * 16
            bidv = bid_v[p, pl.ds(g * 16, 16)] * STYLE
            x = []
            for d in range(STYLE):
                cd = jnp.full((16,), d, jnp.int32)
                wv = plsc.load_gather(rows_v.at[p], [tvec, cd])
                bv = plsc.load_gather(bot_v, [bidv + d])
                x.append(wv * bv)
            s = x[0]
            for d in range(1, STYLE):
                s = s + x[d]
            m = s * (1.0 / STYLE)
            t = [xd - m for xd in x]
            q = t[0] * t[0]
            for d in range(1, STYLE):
                q = q + t[d] * t[d]
            r = _rsqrt(q * (1.0 / STYLE) + EPS)
            oidx = tvec * STYLE
            for d in range(STYLE):
                yd = t[d] * (r * gam[d]) + bet[d]
                plsc.store_scatter(out_v.at[p], [oidx + d], yd)
            return None

        lax.fori_loop(0, GROUPS, group_body, None)
        tok = base_w + ci * CHUNK
        pltpu.sync_copy(out_v.at[p],
                        out_hbm.at[pl.ds(tok * STYLE, CHUNK * STYLE)])

    stage(0, 0)

    def pair_body(k, _):
        stage(2 * k + 1, 1)
        process(2 * k, 0)

        @pl.when(k < N_PAIR - 1)
        def _():
            stage(2 * k + 2, 0)

        process(2 * k + 1, 1)
        return None

    lax.fori_loop(0, N_PAIR, pair_body, None)


@jax.jit
def _run(wids, bids, bottom_flat, word_pad, gamma32, beta32):
    mesh = plsc.VectorSubcoreMesh(core_axis_name="c", subcore_axis_name="s")
    f = functools.partial(
        pl.kernel,
        mesh=mesh,
        out_type=jax.ShapeDtypeStruct((N * STYLE,), jnp.float32),
        scratch_types=[
            pltpu.VMEM((2, CHUNK), jnp.int32),
            pltpu.VMEM((2, CHUNK), jnp.int32),
            pltpu.VMEM((2, CHUNK, WPAD), jnp.float32),
            pltpu.VMEM((2, CHUNK * STYLE), jnp.float32),
            pltpu.VMEM((STYLE * STYLE,), jnp.float32),
            pltpu.VMEM((32,), jnp.float32),
            pltpu.VMEM((32,), jnp.float32),
            pltpu.SemaphoreType.DMA,
            pltpu.SemaphoreType.DMA,
        ],
        compiler_params=pltpu.CompilerParams(
            needs_layout_passes=False, use_tc_tiling_on_sc=False),
    )(_body)
    return f(wids, bids, bottom_flat, word_pad, gamma32, beta32)


def kernel(input_bids, input_wids, bottom_emb, word_emb, gamma, beta):
    wids = input_wids.reshape(-1).astype(jnp.int32)
    bids = input_bids.reshape(-1).astype(jnp.int32)
    word_pad = _pad_table(word_emb)
    bottom_flat = bottom_emb.reshape(-1)
    gam32 = jnp.zeros((32,), jnp.float32).at[:STYLE].set(gamma)
    bet32 = jnp.zeros((32,), jnp.float32).at[:STYLE].set(beta)
    out = _run(wids, bids, bottom_flat, word_pad, gam32, bet32)
    return out.reshape(B, L, STYLE)
